# baseline (speedup 1.0000x reference)
import functools

import jax
import jax.numpy as jnp
from jax import lax
from jax.experimental import pallas as pl
from jax.experimental.pallas import tpu as pltpu
from jax.experimental.pallas import tpu_sc as plsc

N = 10000
E = 320000
D_IN = 128
D_HID = 512

NC = 2
NS = 16
NT = NC * NS
CH = 125
CHP = 128
ECH = E // CH
CPT = ECH // NT
HALF = CPT // 2
NA = N + 8
STRIPE = (N // NS) // 8 * 8
REM = N - NS * STRIPE

_mesh = plsc.VectorSubcoreMesh(core_axis_name="c", subcore_axis_name="s")


@functools.partial(
    pl.kernel,
    out_type=jax.ShapeDtypeStruct((NC, N, 16), jnp.float32),
    mesh=_mesh,
    scratch_types=[
        pltpu.VMEM((CHP, 16), jnp.float32),
        pltpu.VMEM((CPT, CHP), jnp.int32),
        pltpu.VMEM_SHARED((NA, 16), jnp.float32),
        pltpu.SemaphoreType.DMA,
    ],
)
def _degree_kernel(edge_hbm, zeros16_hbm, deg_hbm, ones_v, dst_v, deg_sh, sem):
    c = lax.axis_index("c")
    s = lax.axis_index("s")
    t = c * NS + s

    def init_ones(r, carry):
        ones_v[r, :] = jnp.ones((16,), jnp.float32)
        return carry

    lax.fori_loop(0, CHP, init_ones, 0)

    pltpu.sync_copy(edge_hbm.at[1, pl.ds(t * CPT, CPT)], dst_v)

    pltpu.sync_copy(zeros16_hbm.at[pl.ds(s * STRIPE, STRIPE)],
                    deg_sh.at[pl.ds(s * STRIPE, STRIPE)])

    @pl.when(s == NS - 1)
    def _():
        pltpu.sync_copy(zeros16_hbm.at[pl.ds(NS * STRIPE, REM)],
                        deg_sh.at[pl.ds(NS * STRIPE, REM)])

    plsc.subcore_barrier()

    _W = 8

    def fire(k, carry):
        pltpu.async_copy(ones_v, deg_sh.at[dst_v.at[k]], sem, add=True)

        @pl.when(k >= _W)
        def _():
            pltpu.make_async_copy(ones_v, deg_sh.at[dst_v.at[k - _W]],
                                  sem).wait()

        return carry

    lax.fori_loop(0, CPT, fire, 0)

    def drain(k, carry):
        pltpu.make_async_copy(ones_v, deg_sh.at[dst_v.at[k]], sem).wait()
        return carry

    lax.fori_loop(CPT - _W, CPT, drain, 0)

    plsc.subcore_barrier()
    pltpu.sync_copy(deg_sh.at[pl.ds(s * STRIPE, STRIPE)],
                    deg_hbm.at[c, pl.ds(s * STRIPE, STRIPE)])

    @pl.when(s == NS - 1)
    def _():
        pltpu.sync_copy(deg_sh.at[pl.ds(NS * STRIPE, REM)],
                        deg_hbm.at[c, pl.ds(NS * STRIPE, REM)])


@functools.partial(
    pl.kernel,
    out_type=jax.ShapeDtypeStruct((NC, N, D_IN), jnp.float32),
    mesh=_mesh,
    scratch_types=[
        pltpu.VMEM((HALF, CHP), jnp.int32),
        pltpu.VMEM((HALF, CHP), jnp.int32),
        [pltpu.VMEM((CHP, D_IN), jnp.float32) for _ in range(2)],
        pltpu.VMEM_SHARED((NA, D_IN), jnp.float32),
        [pltpu.SemaphoreType.DMA for _ in range(2)],
        [pltpu.SemaphoreType.DMA for _ in range(2)],
    ],
)
def _scatter_kernel(edge_hbm, y_hbm, zeros_hbm, out_hbm,
                    src_v, dst_v, rows, acc_sh, gsem, ssem):
    c = lax.axis_index("c")
    s = lax.axis_index("s")
    t = c * NS + s

    pltpu.sync_copy(zeros_hbm.at[pl.ds(s * STRIPE, STRIPE)],
                    acc_sh.at[pl.ds(s * STRIPE, STRIPE)])

    @pl.when(s == NS - 1)
    def _():
        pltpu.sync_copy(zeros_hbm.at[pl.ds(NS * STRIPE, REM)],
                        acc_sh.at[pl.ds(NS * STRIPE, REM)])

    plsc.subcore_barrier()

    for st in range(2):
        base = t * CPT + st * HALF
        pltpu.sync_copy(edge_hbm.at[0, pl.ds(base, HALF)], src_v)
        pltpu.sync_copy(edge_hbm.at[1, pl.ds(base, HALF)], dst_v)

        def step(j, carry):
            pltpu.async_copy(y_hbm.at[src_v.at[j]], rows[0], gsem[0]).wait()
            pltpu.sync_copy(rows[0], acc_sh.at[dst_v.at[j]], add=True)
            return carry

        lax.fori_loop(0, HALF, step, 0)

    plsc.subcore_barrier()
    pltpu.sync_copy(acc_sh.at[pl.ds(s * STRIPE, STRIPE)],
                    out_hbm.at[c, pl.ds(s * STRIPE, STRIPE)])

    @pl.when(s == NS - 1)
    def _():
        pltpu.sync_copy(acc_sh.at[pl.ds(NS * STRIPE, REM)],
                        out_hbm.at[c, pl.ds(NS * STRIPE, REM)])


_RB = 1000


def _scale_body(deg_ref, x_ref, y_ref):
    d16 = deg_ref[0] + deg_ref[1]
    deg = jnp.sum(d16, axis=1) * (1.0 / 16.0) + 1.0
    dinv = lax.rsqrt(deg)
    y_ref[...] = x_ref[...] * dinv[:, None]


def _scale(deg16, x):
    return pl.pallas_call(
        _scale_body,
        grid=(N // _RB,),
        in_specs=[
            pl.BlockSpec((NC, _RB, 16), lambda i: (0, i, 0)),
            pl.BlockSpec((_RB, D_IN), lambda i: (i, 0)),
        ],
        out_specs=pl.BlockSpec((_RB, D_IN), lambda i: (i, 0)),
        out_shape=jax.ShapeDtypeStruct((N, D_IN), jnp.float32),
    )(deg16, x)


def _mlp_body(p_ref, y_ref, deg_ref, w1_ref, b1_ref, w2_ref, b2_ref,
              w3_ref, b3_ref, out_ref):
    d16 = deg_ref[0] + deg_ref[1]
    deg = jnp.sum(d16, axis=1) * (1.0 / 16.0) + 1.0
    dinv = lax.rsqrt(deg)
    agg = (p_ref[0] + p_ref[1] + y_ref[...]) * dinv[:, None]
    h = jnp.dot(agg, w1_ref[...], preferred_element_type=jnp.float32,
                precision=lax.Precision.HIGHEST) + b1_ref[...]
    h = jnp.where(h > 0, h, 0.2 * h)
    h = jnp.dot(h, w2_ref[...], preferred_element_type=jnp.float32,
                precision=lax.Precision.HIGHEST) + b2_ref[...]
    h = jnp.where(h > 0, h, 0.2 * h)
    out_ref[...] = jnp.dot(h, w3_ref[...], preferred_element_type=jnp.float32,
                           precision=lax.Precision.HIGHEST) + b3_ref[...]


def _mlp(parts, y, deg16, W1, b1, W2, b2, W3, b3):
    return pl.pallas_call(
        _mlp_body,
        grid=(N // _RB,),
        in_specs=[
            pl.BlockSpec((NC, _RB, D_IN), lambda i: (0, i, 0)),
            pl.BlockSpec((_RB, D_IN), lambda i: (i, 0)),
            pl.BlockSpec((NC, _RB, 16), lambda i: (0, i, 0)),
            pl.BlockSpec((D_IN, D_HID), lambda i: (0, 0)),
            pl.BlockSpec((D_HID,), lambda i: (0,)),
            pl.BlockSpec((D_HID, D_HID), lambda i: (0, 0)),
            pl.BlockSpec((D_HID,), lambda i: (0,)),
            pl.BlockSpec((D_HID, 1), lambda i: (0, 0)),
            pl.BlockSpec((1,), lambda i: (0,)),
        ],
        out_specs=pl.BlockSpec((_RB, 1), lambda i: (i, 0)),
        out_shape=jax.ShapeDtypeStruct((N, 1), jnp.float32),
    )(parts, y, deg16, W1, b1, W2, b2, W3, b3)


def kernel(input_embd, edge_index, W1, b1, W2, b2, W3, b3):
    e3 = edge_index.reshape(2, ECH, CH)
    pad = jnp.stack([jnp.zeros((ECH, CHP - CH), jnp.int32),
                     jnp.full((ECH, CHP - CH), N, jnp.int32)])
    edge3 = jnp.concatenate([e3, pad], axis=2)
    zeros16 = jnp.zeros((N, 16), jnp.float32)
    zeros128 = jnp.zeros((N, D_IN), jnp.float32)
    deg16 = _degree_kernel(edge3, zeros16)
    y = _scale(deg16, input_embd)
    parts = _scatter_kernel(edge3, y, zeros128)
    return _mlp(parts, y, deg16, W1, b1, W2, b2, W3, b3)


# speedup vs baseline: 2.0219x; 2.0219x over previous
import functools

import jax
import jax.numpy as jnp
from jax import lax
from jax.experimental import pallas as pl
from jax.experimental.pallas import tpu as pltpu
from jax.experimental.pallas import tpu_sc as plsc

N = 10000
E = 320000
D_IN = 128
D_HID = 512

NC = 2
NS = 16
NT = NC * NS
CH = 125
CHP = 128
ECH = E // CH
CPT = ECH // NT
HALF = CPT // 2
NA = N + 8
STRIPE = (N // NS) // 8 * 8
REM = N - NS * STRIPE

_mesh = plsc.VectorSubcoreMesh(core_axis_name="c", subcore_axis_name="s")


@functools.partial(
    pl.kernel,
    out_type=jax.ShapeDtypeStruct((NC, N, 16), jnp.float32),
    mesh=_mesh,
    scratch_types=[
        pltpu.VMEM((CHP, 16), jnp.float32),
        pltpu.VMEM((CPT, CHP), jnp.int32),
        pltpu.VMEM_SHARED((NA, 16), jnp.float32),
        pltpu.SemaphoreType.DMA,
    ],
)
def _degree_kernel(edge_hbm, zeros16_hbm, deg_hbm, ones_v, dst_v, deg_sh, sem):
    c = lax.axis_index("c")
    s = lax.axis_index("s")
    t = c * NS + s

    def init_ones(r, carry):
        ones_v[r, :] = jnp.ones((16,), jnp.float32)
        return carry

    lax.fori_loop(0, CHP, init_ones, 0)

    pltpu.sync_copy(edge_hbm.at[1, pl.ds(t * CPT, CPT)], dst_v)

    pltpu.sync_copy(zeros16_hbm.at[pl.ds(s * STRIPE, STRIPE)],
                    deg_sh.at[pl.ds(s * STRIPE, STRIPE)])

    @pl.when(s == NS - 1)
    def _():
        pltpu.sync_copy(zeros16_hbm.at[pl.ds(NS * STRIPE, REM)],
                        deg_sh.at[pl.ds(NS * STRIPE, REM)])

    plsc.subcore_barrier()

    _W = 8

    def fire(k, carry):
        pltpu.async_copy(ones_v, deg_sh.at[dst_v.at[k]], sem, add=True)

        @pl.when(k >= _W)
        def _():
            pltpu.make_async_copy(ones_v, deg_sh.at[dst_v.at[k - _W]],
                                  sem).wait()

        return carry

    lax.fori_loop(0, CPT, fire, 0)

    def drain(k, carry):
        pltpu.make_async_copy(ones_v, deg_sh.at[dst_v.at[k]], sem).wait()
        return carry

    lax.fori_loop(CPT - _W, CPT, drain, 0)

    plsc.subcore_barrier()
    pltpu.sync_copy(deg_sh.at[pl.ds(s * STRIPE, STRIPE)],
                    deg_hbm.at[c, pl.ds(s * STRIPE, STRIPE)])

    @pl.when(s == NS - 1)
    def _():
        pltpu.sync_copy(deg_sh.at[pl.ds(NS * STRIPE, REM)],
                        deg_hbm.at[c, pl.ds(NS * STRIPE, REM)])


EPT = E // NT
CH2 = 128
FULL = EPT // CH2
TAIL = EPT - FULL * CH2


@functools.partial(
    pl.kernel,
    out_type=jax.ShapeDtypeStruct((NC, N, D_IN), jnp.float32),
    mesh=_mesh,
    scratch_types=[
        [pltpu.VMEM((CH2,), jnp.int32) for _ in range(4)],
        [pltpu.VMEM((CH2,), jnp.int32) for _ in range(4)],
        [pltpu.VMEM((CH2, D_IN), jnp.float32) for _ in range(2)],
        pltpu.VMEM((TAIL,), jnp.int32),
        pltpu.VMEM((TAIL,), jnp.int32),
        pltpu.VMEM((TAIL, D_IN), jnp.float32),
        pltpu.VMEM_SHARED((N, D_IN), jnp.float32),
        [pltpu.SemaphoreType.DMA for _ in range(4)],
        [pltpu.SemaphoreType.DMA for _ in range(2)],
        [pltpu.SemaphoreType.DMA for _ in range(2)],
    ],
)
def _scatter_kernel(edge_hbm, y_hbm, zeros_hbm, out_hbm,
                    srcJ, dstJ, rows, srcT, dstT, rowsT, acc_sh,
                    isem, gsem, ssem):
    c = lax.axis_index("c")
    s = lax.axis_index("s")
    t = c * NS + s
    base_e = t * EPT

    pltpu.sync_copy(zeros_hbm.at[pl.ds(s * STRIPE, STRIPE)],
                    acc_sh.at[pl.ds(s * STRIPE, STRIPE)])

    @pl.when(s == NS - 1)
    def _():
        pltpu.sync_copy(zeros_hbm.at[pl.ds(NS * STRIPE, REM)],
                        acc_sh.at[pl.ds(NS * STRIPE, REM)])

    plsc.subcore_barrier()

    def start_idx(k, m):
        pltpu.async_copy(edge_hbm.at[pl.ds(base_e + k * CH2, CH2)],
                         srcJ[m], isem[m])
        pltpu.async_copy(edge_hbm.at[pl.ds(E + base_e + k * CH2, CH2)],
                         dstJ[m], isem[m])

    def wait_idx(k, m):
        pltpu.make_async_copy(edge_hbm.at[pl.ds(base_e + k * CH2, CH2)],
                              srcJ[m], isem[m]).wait()
        pltpu.make_async_copy(edge_hbm.at[pl.ds(E + base_e + k * CH2, CH2)],
                              dstJ[m], isem[m]).wait()

    start_idx(0, 0)
    start_idx(1, 1)

    def step(i, carry):
        for m in range(4):
            k = 4 * i + m
            r = m % 2
            wait_idx(k, m)

            @pl.when(k >= 2)
            def _():
                pltpu.make_async_copy(rows[r], acc_sh.at[dstJ[m]],
                                      ssem[r]).wait()

            pltpu.async_copy(y_hbm.at[srcJ[m]], rows[r], gsem[r])

            @pl.when(k + 2 < FULL)
            def _():
                start_idx(k + 2, (m + 2) % 4)

            m1 = (m + 3) % 4
            r1 = 1 - r

            @pl.when(k >= 1)
            def _():
                pltpu.make_async_copy(y_hbm.at[srcJ[m1]], rows[r1],
                                      gsem[r1]).wait()
                pltpu.async_copy(rows[r1], acc_sh.at[dstJ[m1]], ssem[r1],
                                 add=True)

        return carry

    lax.fori_loop(0, FULL // 4, step, 0)
    for k in (FULL - 2, FULL - 1):
        m = k % 4
        r = m % 2
        wait_idx(k, m)
        pltpu.make_async_copy(rows[r], acc_sh.at[dstJ[m]], ssem[r]).wait()
        pltpu.async_copy(y_hbm.at[srcJ[m]], rows[r], gsem[r])
        m1, r1 = (m + 3) % 4, 1 - r
        pltpu.make_async_copy(y_hbm.at[srcJ[m1]], rows[r1], gsem[r1]).wait()
        pltpu.async_copy(rows[r1], acc_sh.at[dstJ[m1]], ssem[r1], add=True)

    mL = (FULL - 1) % 4
    rL = mL % 2
    pltpu.make_async_copy(y_hbm.at[srcJ[mL]], rows[rL], gsem[rL]).wait()
    pltpu.async_copy(rows[rL], acc_sh.at[dstJ[mL]], ssem[rL], add=True)
    pltpu.make_async_copy(rows[0], acc_sh.at[dstJ[0]], ssem[0]).wait()
    pltpu.make_async_copy(rows[1], acc_sh.at[dstJ[1]], ssem[1]).wait()

    bt = base_e + FULL * CH2
    pltpu.sync_copy(edge_hbm.at[pl.ds(bt, TAIL)], srcT)
    pltpu.sync_copy(edge_hbm.at[pl.ds(E + bt, TAIL)], dstT)
    pltpu.async_copy(y_hbm.at[srcT], rowsT, gsem[0]).wait()
    pltpu.sync_copy(rowsT, acc_sh.at[dstT], add=True)

    plsc.subcore_barrier()
    pltpu.sync_copy(acc_sh.at[pl.ds(s * STRIPE, STRIPE)],
                    out_hbm.at[c, pl.ds(s * STRIPE, STRIPE)])

    @pl.when(s == NS - 1)
    def _():
        pltpu.sync_copy(acc_sh.at[pl.ds(NS * STRIPE, REM)],
                        out_hbm.at[c, pl.ds(NS * STRIPE, REM)])


_RB = 1000


def _scale_body(deg_ref, x_ref, y_ref):
    d16 = deg_ref[0] + deg_ref[1]
    deg = jnp.sum(d16, axis=1) * (1.0 / 16.0) + 1.0
    dinv = lax.rsqrt(deg)
    y_ref[...] = x_ref[...] * dinv[:, None]


def _scale(deg16, x):
    return pl.pallas_call(
        _scale_body,
        grid=(N // _RB,),
        in_specs=[
            pl.BlockSpec((NC, _RB, 16), lambda i: (0, i, 0)),
            pl.BlockSpec((_RB, D_IN), lambda i: (i, 0)),
        ],
        out_specs=pl.BlockSpec((_RB, D_IN), lambda i: (i, 0)),
        out_shape=jax.ShapeDtypeStruct((N, D_IN), jnp.float32),
    )(deg16, x)


def _mlp_body(p_ref, y_ref, deg_ref, w1_ref, b1_ref, w2_ref, b2_ref,
              w3_ref, b3_ref, out_ref):
    d16 = deg_ref[0] + deg_ref[1]
    deg = jnp.sum(d16, axis=1) * (1.0 / 16.0) + 1.0
    dinv = lax.rsqrt(deg)
    agg = (p_ref[0] + p_ref[1] + y_ref[...]) * dinv[:, None]
    h = jnp.dot(agg, w1_ref[...], preferred_element_type=jnp.float32,
                precision=lax.Precision.HIGHEST) + b1_ref[...]
    h = jnp.where(h > 0, h, 0.2 * h)
    h = jnp.dot(h, w2_ref[...], preferred_element_type=jnp.float32,
                precision=lax.Precision.HIGHEST) + b2_ref[...]
    h = jnp.where(h > 0, h, 0.2 * h)
    out_ref[...] = jnp.dot(h, w3_ref[...], preferred_element_type=jnp.float32,
                           precision=lax.Precision.HIGHEST) + b3_ref[...]


def _mlp(parts, y, deg16, W1, b1, W2, b2, W3, b3):
    return pl.pallas_call(
        _mlp_body,
        grid=(N // _RB,),
        in_specs=[
            pl.BlockSpec((NC, _RB, D_IN), lambda i: (0, i, 0)),
            pl.BlockSpec((_RB, D_IN), lambda i: (i, 0)),
            pl.BlockSpec((NC, _RB, 16), lambda i: (0, i, 0)),
            pl.BlockSpec((D_IN, D_HID), lambda i: (0, 0)),
            pl.BlockSpec((D_HID,), lambda i: (0,)),
            pl.BlockSpec((D_HID, D_HID), lambda i: (0, 0)),
            pl.BlockSpec((D_HID,), lambda i: (0,)),
            pl.BlockSpec((D_HID, 1), lambda i: (0, 0)),
            pl.BlockSpec((1,), lambda i: (0,)),
        ],
        out_specs=pl.BlockSpec((_RB, 1), lambda i: (i, 0)),
        out_shape=jax.ShapeDtypeStruct((N, 1), jnp.float32),
    )(parts, y, deg16, W1, b1, W2, b2, W3, b3)


def kernel(input_embd, edge_index, W1, b1, W2, b2, W3, b3):
    e3 = edge_index.reshape(2, ECH, CH)
    pad = jnp.stack([jnp.zeros((ECH, CHP - CH), jnp.int32),
                     jnp.full((ECH, CHP - CH), N, jnp.int32)])
    edge3 = jnp.concatenate([e3, pad], axis=2)
    zeros16 = jnp.zeros((N, 16), jnp.float32)
    zeros128 = jnp.zeros((N, D_IN), jnp.float32)
    deg16 = _degree_kernel(edge3, zeros16)
    y = _scale(deg16, input_embd)
    parts = _scatter_kernel(edge_index.reshape(-1), y, zeros128)
    return _mlp(parts, y, deg16, W1, b1, W2, b2, W3, b3)


# speedup vs baseline: 2.0665x; 1.0221x over previous
import functools

import jax
import jax.numpy as jnp
from jax import lax
from jax.experimental import pallas as pl
from jax.experimental.pallas import tpu as pltpu
from jax.experimental.pallas import tpu_sc as plsc

N = 10000
E = 320000
D_IN = 128
D_HID = 512

NC = 2
NS = 16
NT = NC * NS
STRIPE = (N // NS) // 8 * 8
REM = N - NS * STRIPE

_mesh = plsc.VectorSubcoreMesh(core_axis_name="c", subcore_axis_name="s")


DEPT = E // NT
DCH = 128
DFULL = DEPT // DCH
DTAIL = DEPT - DFULL * DCH


@functools.partial(
    pl.kernel,
    out_type=jax.ShapeDtypeStruct((NC, N, 16), jnp.float32),
    mesh=_mesh,
    scratch_types=[
        pltpu.VMEM((DCH, 16), jnp.float32),
        [pltpu.VMEM((DCH,), jnp.int32) for _ in range(8)],
        pltpu.VMEM((DTAIL,), jnp.int32),
        pltpu.VMEM_SHARED((N, 16), jnp.float32),
        [pltpu.SemaphoreType.DMA for _ in range(8)],
        pltpu.SemaphoreType.DMA,
    ],
)
def _degree_kernel(edge_hbm, zeros16_hbm, deg_hbm, ones_v, dstD, dstT,
                   deg_sh, isem, asem):
    c = lax.axis_index("c")
    s = lax.axis_index("s")
    t = c * NS + s
    base_e = E + t * DEPT

    def init_ones(r, carry):
        ones_v[r, :] = jnp.ones((16,), jnp.float32)
        return carry

    lax.fori_loop(0, DCH, init_ones, 0)

    pltpu.sync_copy(zeros16_hbm.at[pl.ds(s * STRIPE, STRIPE)],
                    deg_sh.at[pl.ds(s * STRIPE, STRIPE)])

    @pl.when(s == NS - 1)
    def _():
        pltpu.sync_copy(zeros16_hbm.at[pl.ds(NS * STRIPE, REM)],
                        deg_sh.at[pl.ds(NS * STRIPE, REM)])

    plsc.subcore_barrier()

    def start_idx(k, m):
        pltpu.async_copy(edge_hbm.at[pl.ds(base_e + k * DCH, DCH)],
                         dstD[m], isem[m])

    for m in range(4):
        start_idx(m, m)

    def step(i, carry):
        for m in range(8):
            k = 8 * i + m
            pltpu.make_async_copy(edge_hbm.at[pl.ds(base_e + k * DCH, DCH)],
                                  dstD[m], isem[m]).wait()
            pltpu.async_copy(ones_v, deg_sh.at[dstD[m]], asem, add=True)

            @pl.when(k >= 4)
            def _():
                pltpu.make_async_copy(ones_v, deg_sh.at[dstD[(m + 4) % 8]],
                                      asem).wait()

            @pl.when(k + 4 < DFULL)
            def _():
                start_idx(k + 4, (m + 4) % 8)

        return carry

    lax.fori_loop(0, DFULL // 8, step, 0)
    for k in range(8 * (DFULL // 8), DFULL):
        m = k % 8
        pltpu.make_async_copy(edge_hbm.at[pl.ds(base_e + k * DCH, DCH)],
                              dstD[m], isem[m]).wait()
        pltpu.async_copy(ones_v, deg_sh.at[dstD[m]], asem, add=True)
        pltpu.make_async_copy(ones_v, deg_sh.at[dstD[(m + 4) % 8]],
                              asem).wait()
        if k + 4 < DFULL:
            start_idx(k + 4, (m + 4) % 8)

    for k in range(DFULL - 4, DFULL):
        pltpu.make_async_copy(ones_v, deg_sh.at[dstD[k % 8]], asem).wait()

    pltpu.sync_copy(edge_hbm.at[pl.ds(base_e + DFULL * DCH, DTAIL)], dstT)
    pltpu.sync_copy(ones_v.at[pl.ds(0, DTAIL)], deg_sh.at[dstT], add=True)

    plsc.subcore_barrier()
    pltpu.sync_copy(deg_sh.at[pl.ds(s * STRIPE, STRIPE)],
                    deg_hbm.at[c, pl.ds(s * STRIPE, STRIPE)])

    @pl.when(s == NS - 1)
    def _():
        pltpu.sync_copy(deg_sh.at[pl.ds(NS * STRIPE, REM)],
                        deg_hbm.at[c, pl.ds(NS * STRIPE, REM)])


EPT = E // NT
CH2 = 128
FULL = EPT // CH2
TAIL = EPT - FULL * CH2


@functools.partial(
    pl.kernel,
    out_type=jax.ShapeDtypeStruct((NC, N, D_IN), jnp.float32),
    mesh=_mesh,
    scratch_types=[
        [pltpu.VMEM((CH2,), jnp.int32) for _ in range(4)],
        [pltpu.VMEM((CH2,), jnp.int32) for _ in range(4)],
        [pltpu.VMEM((CH2, D_IN), jnp.float32) for _ in range(2)],
        pltpu.VMEM((TAIL,), jnp.int32),
        pltpu.VMEM((TAIL,), jnp.int32),
        pltpu.VMEM((TAIL, D_IN), jnp.float32),
        pltpu.VMEM_SHARED((N, D_IN), jnp.float32),
        [pltpu.SemaphoreType.DMA for _ in range(4)],
        [pltpu.SemaphoreType.DMA for _ in range(2)],
        [pltpu.SemaphoreType.DMA for _ in range(2)],
    ],
)
def _scatter_kernel(edge_hbm, y_hbm, zeros_hbm, out_hbm,
                    srcJ, dstJ, rows, srcT, dstT, rowsT, acc_sh,
                    isem, gsem, ssem):
    c = lax.axis_index("c")
    s = lax.axis_index("s")
    t = c * NS + s
    base_e = t * EPT

    pltpu.sync_copy(zeros_hbm.at[pl.ds(s * STRIPE, STRIPE)],
                    acc_sh.at[pl.ds(s * STRIPE, STRIPE)])

    @pl.when(s == NS - 1)
    def _():
        pltpu.sync_copy(zeros_hbm.at[pl.ds(NS * STRIPE, REM)],
                        acc_sh.at[pl.ds(NS * STRIPE, REM)])

    plsc.subcore_barrier()

    def start_idx(k, m):
        pltpu.async_copy(edge_hbm.at[pl.ds(base_e + k * CH2, CH2)],
                         srcJ[m], isem[m])
        pltpu.async_copy(edge_hbm.at[pl.ds(E + base_e + k * CH2, CH2)],
                         dstJ[m], isem[m])

    def wait_idx(k, m):
        pltpu.make_async_copy(edge_hbm.at[pl.ds(base_e + k * CH2, CH2)],
                              srcJ[m], isem[m]).wait()
        pltpu.make_async_copy(edge_hbm.at[pl.ds(E + base_e + k * CH2, CH2)],
                              dstJ[m], isem[m]).wait()

    start_idx(0, 0)
    start_idx(1, 1)

    def step(i, carry):
        for m in range(4):
            k = 4 * i + m
            r = m % 2
            wait_idx(k, m)

            @pl.when(k >= 2)
            def _():
                pltpu.make_async_copy(rows[r], acc_sh.at[dstJ[m]],
                                      ssem[r]).wait()

            pltpu.async_copy(y_hbm.at[srcJ[m]], rows[r], gsem[r])

            @pl.when(k + 2 < FULL)
            def _():
                start_idx(k + 2, (m + 2) % 4)

            m1 = (m + 3) % 4
            r1 = 1 - r

            @pl.when(k >= 1)
            def _():
                pltpu.make_async_copy(y_hbm.at[srcJ[m1]], rows[r1],
                                      gsem[r1]).wait()
                pltpu.async_copy(rows[r1], acc_sh.at[dstJ[m1]], ssem[r1],
                                 add=True)

        return carry

    lax.fori_loop(0, FULL // 4, step, 0)
    for k in (FULL - 2, FULL - 1):
        m = k % 4
        r = m % 2
        wait_idx(k, m)
        pltpu.make_async_copy(rows[r], acc_sh.at[dstJ[m]], ssem[r]).wait()
        pltpu.async_copy(y_hbm.at[srcJ[m]], rows[r], gsem[r])
        m1, r1 = (m + 3) % 4, 1 - r
        pltpu.make_async_copy(y_hbm.at[srcJ[m1]], rows[r1], gsem[r1]).wait()
        pltpu.async_copy(rows[r1], acc_sh.at[dstJ[m1]], ssem[r1], add=True)

    mL = (FULL - 1) % 4
    rL = mL % 2
    pltpu.make_async_copy(y_hbm.at[srcJ[mL]], rows[rL], gsem[rL]).wait()
    pltpu.async_copy(rows[rL], acc_sh.at[dstJ[mL]], ssem[rL], add=True)
    pltpu.make_async_copy(rows[0], acc_sh.at[dstJ[0]], ssem[0]).wait()
    pltpu.make_async_copy(rows[1], acc_sh.at[dstJ[1]], ssem[1]).wait()

    bt = base_e + FULL * CH2
    pltpu.sync_copy(edge_hbm.at[pl.ds(bt, TAIL)], srcT)
    pltpu.sync_copy(edge_hbm.at[pl.ds(E + bt, TAIL)], dstT)
    pltpu.async_copy(y_hbm.at[srcT], rowsT, gsem[0]).wait()
    pltpu.sync_copy(rowsT, acc_sh.at[dstT], add=True)

    plsc.subcore_barrier()
    pltpu.sync_copy(acc_sh.at[pl.ds(s * STRIPE, STRIPE)],
                    out_hbm.at[c, pl.ds(s * STRIPE, STRIPE)])

    @pl.when(s == NS - 1)
    def _():
        pltpu.sync_copy(acc_sh.at[pl.ds(NS * STRIPE, REM)],
                        out_hbm.at[c, pl.ds(NS * STRIPE, REM)])


_RB = 1000


def _scale_body(deg_ref, x_ref, y_ref):
    d16 = deg_ref[0] + deg_ref[1]
    deg = jnp.sum(d16, axis=1) * (1.0 / 16.0) + 1.0
    dinv = lax.rsqrt(deg)
    y_ref[...] = x_ref[...] * dinv[:, None]


def _scale(deg16, x):
    return pl.pallas_call(
        _scale_body,
        grid=(N // _RB,),
        in_specs=[
            pl.BlockSpec((NC, _RB, 16), lambda i: (0, i, 0)),
            pl.BlockSpec((_RB, D_IN), lambda i: (i, 0)),
        ],
        out_specs=pl.BlockSpec((_RB, D_IN), lambda i: (i, 0)),
        out_shape=jax.ShapeDtypeStruct((N, D_IN), jnp.float32),
    )(deg16, x)


def _mlp_body(p_ref, y_ref, deg_ref, w1_ref, b1_ref, w2_ref, b2_ref,
              w3_ref, b3_ref, out_ref):
    d16 = deg_ref[0] + deg_ref[1]
    deg = jnp.sum(d16, axis=1) * (1.0 / 16.0) + 1.0
    dinv = lax.rsqrt(deg)
    agg = (p_ref[0] + p_ref[1] + y_ref[...]) * dinv[:, None]
    h = jnp.dot(agg, w1_ref[...], preferred_element_type=jnp.float32,
                precision=lax.Precision.HIGHEST) + b1_ref[...]
    h = jnp.where(h > 0, h, 0.2 * h)
    h = jnp.dot(h, w2_ref[...], preferred_element_type=jnp.float32,
                precision=lax.Precision.HIGHEST) + b2_ref[...]
    h = jnp.where(h > 0, h, 0.2 * h)
    out_ref[...] = jnp.dot(h, w3_ref[...], preferred_element_type=jnp.float32,
                           precision=lax.Precision.HIGHEST) + b3_ref[...]


def _mlp(parts, y, deg16, W1, b1, W2, b2, W3, b3):
    return pl.pallas_call(
        _mlp_body,
        grid=(N // _RB,),
        in_specs=[
            pl.BlockSpec((NC, _RB, D_IN), lambda i: (0, i, 0)),
            pl.BlockSpec((_RB, D_IN), lambda i: (i, 0)),
            pl.BlockSpec((NC, _RB, 16), lambda i: (0, i, 0)),
            pl.BlockSpec((D_IN, D_HID), lambda i: (0, 0)),
            pl.BlockSpec((D_HID,), lambda i: (0,)),
            pl.BlockSpec((D_HID, D_HID), lambda i: (0, 0)),
            pl.BlockSpec((D_HID,), lambda i: (0,)),
            pl.BlockSpec((D_HID, 1), lambda i: (0, 0)),
            pl.BlockSpec((1,), lambda i: (0,)),
        ],
        out_specs=pl.BlockSpec((_RB, 1), lambda i: (i, 0)),
        out_shape=jax.ShapeDtypeStruct((N, 1), jnp.float32),
    )(parts, y, deg16, W1, b1, W2, b2, W3, b3)


def kernel(input_embd, edge_index, W1, b1, W2, b2, W3, b3):
    edge_flat = edge_index.reshape(-1)
    zeros16 = jnp.zeros((N, 16), jnp.float32)
    zeros128 = jnp.zeros((N, D_IN), jnp.float32)
    deg16 = _degree_kernel(edge_flat, zeros16)
    y = _scale(deg16, input_embd)
    parts = _scatter_kernel(edge_flat, y, zeros128)
    return _mlp(parts, y, deg16, W1, b1, W2, b2, W3, b3)


# speedup vs baseline: 2.7431x; 1.3274x over previous
import functools

import jax
import jax.numpy as jnp
from jax import lax
from jax.experimental import pallas as pl
from jax.experimental.pallas import tpu as pltpu
from jax.experimental.pallas import tpu_sc as plsc

N = 10000
E = 320000
D_IN = 128
D_HID = 512

NC = 2
NS = 16
NT = NC * NS
STRIPE = (N // NS) // 8 * 8
REM = N - NS * STRIPE

_mesh = plsc.VectorSubcoreMesh(core_axis_name="c", subcore_axis_name="s")


DEPT = E // NT
DCH = 128
DFULL = DEPT // DCH
DTAIL = DEPT - DFULL * DCH


@functools.partial(
    pl.kernel,
    out_type=jax.ShapeDtypeStruct((NC, N, 16), jnp.float32),
    mesh=_mesh,
    scratch_types=[
        pltpu.VMEM((DCH, 16), jnp.float32),
        [pltpu.VMEM((DCH,), jnp.int32) for _ in range(8)],
        pltpu.VMEM((DTAIL,), jnp.int32),
        pltpu.VMEM_SHARED((N, 16), jnp.float32),
        [pltpu.SemaphoreType.DMA for _ in range(8)],
        pltpu.SemaphoreType.DMA,
    ],
)
def _degree_kernel(edge_hbm, zeros16_hbm, deg_hbm, ones_v, dstD, dstT,
                   deg_sh, isem, asem):
    c = lax.axis_index("c")
    s = lax.axis_index("s")
    t = c * NS + s
    base_e = E + t * DEPT

    def init_ones(r, carry):
        ones_v[r, :] = jnp.ones((16,), jnp.float32)
        return carry

    lax.fori_loop(0, DCH, init_ones, 0)

    pltpu.sync_copy(zeros16_hbm.at[pl.ds(s * STRIPE, STRIPE)],
                    deg_sh.at[pl.ds(s * STRIPE, STRIPE)])

    @pl.when(s == NS - 1)
    def _():
        pltpu.sync_copy(zeros16_hbm.at[pl.ds(NS * STRIPE, REM)],
                        deg_sh.at[pl.ds(NS * STRIPE, REM)])

    plsc.subcore_barrier()

    def start_idx(k, m):
        pltpu.async_copy(edge_hbm.at[pl.ds(base_e + k * DCH, DCH)],
                         dstD[m], isem[m])

    for m in range(4):
        start_idx(m, m)

    def step(i, carry):
        for m in range(8):
            k = 8 * i + m
            pltpu.make_async_copy(edge_hbm.at[pl.ds(base_e + k * DCH, DCH)],
                                  dstD[m], isem[m]).wait()
            pltpu.async_copy(ones_v, deg_sh.at[dstD[m]], asem, add=True)

            @pl.when(k >= 4)
            def _():
                pltpu.make_async_copy(ones_v, deg_sh.at[dstD[(m + 4) % 8]],
                                      asem).wait()

            @pl.when(k + 4 < DFULL)
            def _():
                start_idx(k + 4, (m + 4) % 8)

        return carry

    lax.fori_loop(0, DFULL // 8, step, 0)
    for k in range(8 * (DFULL // 8), DFULL):
        m = k % 8
        pltpu.make_async_copy(edge_hbm.at[pl.ds(base_e + k * DCH, DCH)],
                              dstD[m], isem[m]).wait()
        pltpu.async_copy(ones_v, deg_sh.at[dstD[m]], asem, add=True)
        pltpu.make_async_copy(ones_v, deg_sh.at[dstD[(m + 4) % 8]],
                              asem).wait()
        if k + 4 < DFULL:
            start_idx(k + 4, (m + 4) % 8)

    for k in range(DFULL - 4, DFULL):
        pltpu.make_async_copy(ones_v, deg_sh.at[dstD[k % 8]], asem).wait()

    pltpu.sync_copy(edge_hbm.at[pl.ds(base_e + DFULL * DCH, DTAIL)], dstT)
    pltpu.sync_copy(ones_v.at[pl.ds(0, DTAIL)], deg_sh.at[dstT], add=True)

    plsc.subcore_barrier()
    pltpu.sync_copy(deg_sh.at[pl.ds(s * STRIPE, STRIPE)],
                    deg_hbm.at[c, pl.ds(s * STRIPE, STRIPE)])

    @pl.when(s == NS - 1)
    def _():
        pltpu.sync_copy(deg_sh.at[pl.ds(NS * STRIPE, REM)],
                        deg_hbm.at[c, pl.ds(NS * STRIPE, REM)])


EPT = E // NT
CH2 = 128
FULL = EPT // CH2
TAIL = EPT - FULL * CH2


@functools.partial(
    pl.kernel,
    out_type=jax.ShapeDtypeStruct((NC, N, D_IN), jnp.float32),
    mesh=_mesh,
    scratch_types=[
        [pltpu.VMEM((CH2,), jnp.int32) for _ in range(4)],
        [pltpu.VMEM((CH2,), jnp.int32) for _ in range(4)],
        [pltpu.VMEM((CH2, D_IN), jnp.float32) for _ in range(2)],
        pltpu.VMEM((TAIL,), jnp.int32),
        pltpu.VMEM((TAIL,), jnp.int32),
        pltpu.VMEM((TAIL, D_IN), jnp.float32),
        pltpu.VMEM_SHARED((N, D_IN), jnp.float32),
        [pltpu.SemaphoreType.DMA for _ in range(4)],
        [pltpu.SemaphoreType.DMA for _ in range(2)],
        [pltpu.SemaphoreType.DMA for _ in range(2)],
    ],
)
def _scatter_kernel(edge_hbm, y_hbm, zeros_hbm, out_hbm,
                    srcJ, dstJ, rows, srcT, dstT, rowsT, acc_sh,
                    isem, gsem, ssem):
    c = lax.axis_index("c")
    s = lax.axis_index("s")
    t = c * NS + s
    base_e = t * EPT

    pltpu.sync_copy(zeros_hbm.at[pl.ds(s * STRIPE, STRIPE)],
                    acc_sh.at[pl.ds(s * STRIPE, STRIPE)])

    @pl.when(s == NS - 1)
    def _():
        pltpu.sync_copy(zeros_hbm.at[pl.ds(NS * STRIPE, REM)],
                        acc_sh.at[pl.ds(NS * STRIPE, REM)])

    plsc.subcore_barrier()

    def start_idx(k, m):
        pltpu.async_copy(edge_hbm.at[pl.ds(base_e + k * CH2, CH2)],
                         srcJ[m], isem[m])
        pltpu.async_copy(edge_hbm.at[pl.ds(E + base_e + k * CH2, CH2)],
                         dstJ[m], isem[m])

    def wait_idx(k, m):
        pltpu.make_async_copy(edge_hbm.at[pl.ds(base_e + k * CH2, CH2)],
                              srcJ[m], isem[m]).wait()
        pltpu.make_async_copy(edge_hbm.at[pl.ds(E + base_e + k * CH2, CH2)],
                              dstJ[m], isem[m]).wait()

    start_idx(0, 0)
    start_idx(1, 1)

    def step(i, carry):
        for m in range(4):
            k = 4 * i + m
            r = m % 2
            wait_idx(k, m)

            @pl.when(k >= 2)
            def _():
                pltpu.make_async_copy(rows[r], acc_sh.at[dstJ[m]],
                                      ssem[r]).wait()

            pltpu.async_copy(y_hbm.at[srcJ[m]], rows[r], gsem[r])

            @pl.when(k + 2 < FULL)
            def _():
                start_idx(k + 2, (m + 2) % 4)

            m1 = (m + 3) % 4
            r1 = 1 - r

            @pl.when(k >= 1)
            def _():
                pltpu.make_async_copy(y_hbm.at[srcJ[m1]], rows[r1],
                                      gsem[r1]).wait()
                pltpu.async_copy(rows[r1], acc_sh.at[dstJ[m1]], ssem[r1],
                                 add=True)

        return carry

    lax.fori_loop(0, FULL // 4, step, 0)
    for k in (FULL - 2, FULL - 1):
        m = k % 4
        r = m % 2
        wait_idx(k, m)
        pltpu.make_async_copy(rows[r], acc_sh.at[dstJ[m]], ssem[r]).wait()
        pltpu.async_copy(y_hbm.at[srcJ[m]], rows[r], gsem[r])
        m1, r1 = (m + 3) % 4, 1 - r
        pltpu.make_async_copy(y_hbm.at[srcJ[m1]], rows[r1], gsem[r1]).wait()
        pltpu.async_copy(rows[r1], acc_sh.at[dstJ[m1]], ssem[r1], add=True)

    mL = (FULL - 1) % 4
    rL = mL % 2
    pltpu.make_async_copy(y_hbm.at[srcJ[mL]], rows[rL], gsem[rL]).wait()
    pltpu.async_copy(rows[rL], acc_sh.at[dstJ[mL]], ssem[rL], add=True)
    pltpu.make_async_copy(rows[0], acc_sh.at[dstJ[0]], ssem[0]).wait()
    pltpu.make_async_copy(rows[1], acc_sh.at[dstJ[1]], ssem[1]).wait()

    bt = base_e + FULL * CH2
    pltpu.sync_copy(edge_hbm.at[pl.ds(bt, TAIL)], srcT)
    pltpu.sync_copy(edge_hbm.at[pl.ds(E + bt, TAIL)], dstT)
    pltpu.async_copy(y_hbm.at[srcT], rowsT, gsem[0]).wait()
    pltpu.sync_copy(rowsT, acc_sh.at[dstT], add=True)

    plsc.subcore_barrier()
    pltpu.sync_copy(acc_sh.at[pl.ds(s * STRIPE, STRIPE)],
                    out_hbm.at[c, pl.ds(s * STRIPE, STRIPE)])

    @pl.when(s == NS - 1)
    def _():
        pltpu.sync_copy(acc_sh.at[pl.ds(NS * STRIPE, REM)],
                        out_hbm.at[c, pl.ds(NS * STRIPE, REM)])


_RB = 1000


def _scale_body(deg_ref, x_ref, y_ref):
    d16 = deg_ref[0] + deg_ref[1]
    deg = jnp.sum(d16, axis=1) * (1.0 / 16.0) + 1.0
    dinv = lax.rsqrt(deg)
    y_ref[...] = x_ref[...] * dinv[:, None]


def _scale(deg16, x):
    return pl.pallas_call(
        _scale_body,
        grid=(N // _RB,),
        in_specs=[
            pl.BlockSpec((NC, _RB, 16), lambda i: (0, i, 0)),
            pl.BlockSpec((_RB, D_IN), lambda i: (i, 0)),
        ],
        out_specs=pl.BlockSpec((_RB, D_IN), lambda i: (i, 0)),
        out_shape=jax.ShapeDtypeStruct((N, D_IN), jnp.float32),
    )(deg16, x)


def _mlp_body(p_ref, y_ref, deg_ref, w1_ref, b1_ref, w2_ref, b2_ref,
              w3_ref, b3_ref, out_ref):
    d16 = deg_ref[0] + deg_ref[1]
    deg = jnp.sum(d16, axis=1) * (1.0 / 16.0) + 1.0
    dinv = lax.rsqrt(deg)
    agg = (p_ref[0] + p_ref[1] + y_ref[...]) * dinv[:, None]
    h = jnp.dot(agg, w1_ref[...], preferred_element_type=jnp.float32) + b1_ref[...]
    h = jnp.where(h > 0, h, 0.2 * h)
    h = jnp.dot(h, w2_ref[...], preferred_element_type=jnp.float32) + b2_ref[...]
    h = jnp.where(h > 0, h, 0.2 * h)
    out_ref[...] = jnp.dot(h, w3_ref[...], preferred_element_type=jnp.float32) + b3_ref[...]


def _mlp(parts, y, deg16, W1, b1, W2, b2, W3, b3):
    return pl.pallas_call(
        _mlp_body,
        grid=(N // _RB,),
        in_specs=[
            pl.BlockSpec((NC, _RB, D_IN), lambda i: (0, i, 0)),
            pl.BlockSpec((_RB, D_IN), lambda i: (i, 0)),
            pl.BlockSpec((NC, _RB, 16), lambda i: (0, i, 0)),
            pl.BlockSpec((D_IN, D_HID), lambda i: (0, 0)),
            pl.BlockSpec((D_HID,), lambda i: (0,)),
            pl.BlockSpec((D_HID, D_HID), lambda i: (0, 0)),
            pl.BlockSpec((D_HID,), lambda i: (0,)),
            pl.BlockSpec((D_HID, 1), lambda i: (0, 0)),
            pl.BlockSpec((1,), lambda i: (0,)),
        ],
        out_specs=pl.BlockSpec((_RB, 1), lambda i: (i, 0)),
        out_shape=jax.ShapeDtypeStruct((N, 1), jnp.float32),
    )(parts, y, deg16, W1, b1, W2, b2, W3, b3)


def kernel(input_embd, edge_index, W1, b1, W2, b2, W3, b3):
    edge_flat = edge_index.reshape(-1)
    zeros16 = jnp.zeros((N, 16), jnp.float32)
    zeros128 = jnp.zeros((N, D_IN), jnp.float32)
    deg16 = _degree_kernel(edge_flat, zeros16)
    y = _scale(deg16, input_embd)
    parts = _scatter_kernel(edge_flat, y, zeros128)
    return _mlp(parts, y, deg16, W1, b1, W2, b2, W3, b3)


# speedup vs baseline: 2.9263x; 1.0668x over previous
import functools

import jax
import jax.numpy as jnp
from jax import lax
from jax.experimental import pallas as pl
from jax.experimental.pallas import tpu as pltpu
from jax.experimental.pallas import tpu_sc as plsc

N = 10000
E = 320000
D_IN = 128
D_HID = 512

NC = 2
NS = 16
NT = NC * NS
STRIPE = (N // NS) // 8 * 8
REM = N - NS * STRIPE

_mesh = plsc.VectorSubcoreMesh(core_axis_name="c", subcore_axis_name="s")


DEPT = E // NT
DCH = 128
DFULL = DEPT // DCH
DTAIL = DEPT - DFULL * DCH


@functools.partial(
    pl.kernel,
    out_type=jax.ShapeDtypeStruct((NC, N, 16), jnp.float32),
    mesh=_mesh,
    scratch_types=[
        pltpu.VMEM((DCH, 16), jnp.float32),
        [pltpu.VMEM((DCH,), jnp.int32) for _ in range(8)],
        pltpu.VMEM((DTAIL,), jnp.int32),
        pltpu.VMEM_SHARED((N, 16), jnp.float32),
        [pltpu.SemaphoreType.DMA for _ in range(8)],
        pltpu.SemaphoreType.DMA,
    ],
)
def _degree_kernel(edge_hbm, zeros16_hbm, deg_hbm, ones_v, dstD, dstT,
                   deg_sh, isem, asem):
    c = lax.axis_index("c")
    s = lax.axis_index("s")
    t = c * NS + s
    base_e = E + t * DEPT

    def init_ones(r, carry):
        ones_v[r, :] = jnp.ones((16,), jnp.float32)
        return carry

    lax.fori_loop(0, DCH, init_ones, 0)

    pltpu.sync_copy(zeros16_hbm.at[pl.ds(s * STRIPE, STRIPE)],
                    deg_sh.at[pl.ds(s * STRIPE, STRIPE)])

    @pl.when(s == NS - 1)
    def _():
        pltpu.sync_copy(zeros16_hbm.at[pl.ds(NS * STRIPE, REM)],
                        deg_sh.at[pl.ds(NS * STRIPE, REM)])

    plsc.subcore_barrier()

    def start_idx(k, m):
        pltpu.async_copy(edge_hbm.at[pl.ds(base_e + k * DCH, DCH)],
                         dstD[m], isem[m])

    for m in range(4):
        start_idx(m, m)

    def step(i, carry):
        for m in range(8):
            k = 8 * i + m
            pltpu.make_async_copy(edge_hbm.at[pl.ds(base_e + k * DCH, DCH)],
                                  dstD[m], isem[m]).wait()
            pltpu.async_copy(ones_v, deg_sh.at[dstD[m]], asem, add=True)

            @pl.when(k >= 4)
            def _():
                pltpu.make_async_copy(ones_v, deg_sh.at[dstD[(m + 4) % 8]],
                                      asem).wait()

            @pl.when(k + 4 < DFULL)
            def _():
                start_idx(k + 4, (m + 4) % 8)

        return carry

    lax.fori_loop(0, DFULL // 8, step, 0)
    for k in range(8 * (DFULL // 8), DFULL):
        m = k % 8
        pltpu.make_async_copy(edge_hbm.at[pl.ds(base_e + k * DCH, DCH)],
                              dstD[m], isem[m]).wait()
        pltpu.async_copy(ones_v, deg_sh.at[dstD[m]], asem, add=True)
        pltpu.make_async_copy(ones_v, deg_sh.at[dstD[(m + 4) % 8]],
                              asem).wait()
        if k + 4 < DFULL:
            start_idx(k + 4, (m + 4) % 8)

    for k in range(DFULL - 4, DFULL):
        pltpu.make_async_copy(ones_v, deg_sh.at[dstD[k % 8]], asem).wait()

    pltpu.sync_copy(edge_hbm.at[pl.ds(base_e + DFULL * DCH, DTAIL)], dstT)
    pltpu.sync_copy(ones_v.at[pl.ds(0, DTAIL)], deg_sh.at[dstT], add=True)

    plsc.subcore_barrier()
    pltpu.sync_copy(deg_sh.at[pl.ds(s * STRIPE, STRIPE)],
                    deg_hbm.at[c, pl.ds(s * STRIPE, STRIPE)])

    @pl.when(s == NS - 1)
    def _():
        pltpu.sync_copy(deg_sh.at[pl.ds(NS * STRIPE, REM)],
                        deg_hbm.at[c, pl.ds(NS * STRIPE, REM)])


EPT = E // NT
CH2 = 104
FULL = EPT // CH2
TAIL = EPT - FULL * CH2
RI = 6
RR = 3


@functools.partial(
    pl.kernel,
    out_type=jax.ShapeDtypeStruct((NC, N, D_IN), jnp.float32),
    mesh=_mesh,
    scratch_types=[
        [pltpu.VMEM((CH2,), jnp.int32) for _ in range(RI)],
        [pltpu.VMEM((CH2,), jnp.int32) for _ in range(RI)],
        [pltpu.VMEM((CH2, D_IN), jnp.float32) for _ in range(RR)],
        pltpu.VMEM((TAIL,), jnp.int32),
        pltpu.VMEM((TAIL,), jnp.int32),
        pltpu.VMEM((TAIL, D_IN), jnp.float32),
        pltpu.VMEM_SHARED((N, D_IN), jnp.float32),
        [pltpu.SemaphoreType.DMA for _ in range(RI)],
        [pltpu.SemaphoreType.DMA for _ in range(RR)],
        [pltpu.SemaphoreType.DMA for _ in range(RR)],
    ],
)
def _scatter_kernel(edge_hbm, y_hbm, zeros_hbm, out_hbm,
                    srcJ, dstJ, rows, srcT, dstT, rowsT, acc_sh,
                    isem, gsem, ssem):
    c = lax.axis_index("c")
    s = lax.axis_index("s")
    t = c * NS + s
    base_e = t * EPT

    pltpu.sync_copy(zeros_hbm.at[pl.ds(s * STRIPE, STRIPE)],
                    acc_sh.at[pl.ds(s * STRIPE, STRIPE)])

    @pl.when(s == NS - 1)
    def _():
        pltpu.sync_copy(zeros_hbm.at[pl.ds(NS * STRIPE, REM)],
                        acc_sh.at[pl.ds(NS * STRIPE, REM)])

    plsc.subcore_barrier()

    def start_idx(k, m):
        pltpu.async_copy(edge_hbm.at[pl.ds(base_e + k * CH2, CH2)],
                         srcJ[m], isem[m])
        pltpu.async_copy(edge_hbm.at[pl.ds(E + base_e + k * CH2, CH2)],
                         dstJ[m], isem[m])

    def wait_idx(k, m):
        pltpu.make_async_copy(edge_hbm.at[pl.ds(base_e + k * CH2, CH2)],
                              srcJ[m], isem[m]).wait()
        pltpu.make_async_copy(edge_hbm.at[pl.ds(E + base_e + k * CH2, CH2)],
                              dstJ[m], isem[m]).wait()

    def wait_scatter(r, m):
        pltpu.make_async_copy(rows[r], acc_sh.at[dstJ[m]], ssem[r]).wait()

    def issue_scatter(r, m):
        pltpu.async_copy(rows[r], acc_sh.at[dstJ[m]], ssem[r], add=True)

    def wait_gather(r, m):
        pltpu.make_async_copy(y_hbm.at[srcJ[m]], rows[r], gsem[r]).wait()

    for k0 in range(3):
        start_idx(k0, k0)

    def step(i, carry):
        for m in range(RI):
            k = RI * i + m
            r = m % RR
            wait_idx(k, m)

            @pl.when(k >= 3)
            def _():
                wait_scatter(r, m)

            pltpu.async_copy(y_hbm.at[srcJ[m]], rows[r], gsem[r])

            @pl.when(k + 3 < FULL)
            def _():
                start_idx(k + 3, (m + 3) % RI)

            m2 = (m + 4) % RI
            r2 = (m + 1) % RR

            @pl.when(k >= 2)
            def _():
                wait_gather(r2, m2)
                issue_scatter(r2, m2)

        return carry

    lax.fori_loop(0, FULL // RI, step, 0)

    for k in (FULL - 2, FULL - 1):
        m, r = k % RI, k % RR
        wait_gather(r, m)
        issue_scatter(r, m)
    for k in (FULL - 3, FULL - 2, FULL - 1):
        wait_scatter(k % RR, k % RI)

    bt = base_e + FULL * CH2
    pltpu.sync_copy(edge_hbm.at[pl.ds(bt, TAIL)], srcT)
    pltpu.sync_copy(edge_hbm.at[pl.ds(E + bt, TAIL)], dstT)
    pltpu.async_copy(y_hbm.at[srcT], rowsT, gsem[0]).wait()
    pltpu.sync_copy(rowsT, acc_sh.at[dstT], add=True)

    plsc.subcore_barrier()
    pltpu.sync_copy(acc_sh.at[pl.ds(s * STRIPE, STRIPE)],
                    out_hbm.at[c, pl.ds(s * STRIPE, STRIPE)])

    @pl.when(s == NS - 1)
    def _():
        pltpu.sync_copy(acc_sh.at[pl.ds(NS * STRIPE, REM)],
                        out_hbm.at[c, pl.ds(NS * STRIPE, REM)])


_RB = 1000


def _scale_body(deg_ref, x_ref, y_ref):
    d16 = deg_ref[0] + deg_ref[1]
    deg = jnp.sum(d16, axis=1) * (1.0 / 16.0) + 1.0
    dinv = lax.rsqrt(deg)
    y_ref[...] = x_ref[...] * dinv[:, None]


def _scale(deg16, x):
    return pl.pallas_call(
        _scale_body,
        grid=(N // _RB,),
        in_specs=[
            pl.BlockSpec((NC, _RB, 16), lambda i: (0, i, 0)),
            pl.BlockSpec((_RB, D_IN), lambda i: (i, 0)),
        ],
        out_specs=pl.BlockSpec((_RB, D_IN), lambda i: (i, 0)),
        out_shape=jax.ShapeDtypeStruct((N, D_IN), jnp.float32),
    )(deg16, x)


def _mlp_body(p_ref, y_ref, deg_ref, w1_ref, b1_ref, w2_ref, b2_ref,
              w3_ref, b3_ref, out_ref):
    d16 = deg_ref[0] + deg_ref[1]
    deg = jnp.sum(d16, axis=1) * (1.0 / 16.0) + 1.0
    dinv = lax.rsqrt(deg)
    agg = (p_ref[0] + p_ref[1] + y_ref[...]) * dinv[:, None]
    h = jnp.dot(agg, w1_ref[...], preferred_element_type=jnp.float32) + b1_ref[...]
    h = jnp.where(h > 0, h, 0.2 * h)
    h = jnp.dot(h, w2_ref[...], preferred_element_type=jnp.float32) + b2_ref[...]
    h = jnp.where(h > 0, h, 0.2 * h)
    out_ref[...] = jnp.dot(h, w3_ref[...], preferred_element_type=jnp.float32) + b3_ref[...]


def _mlp(parts, y, deg16, W1, b1, W2, b2, W3, b3):
    return pl.pallas_call(
        _mlp_body,
        grid=(N // _RB,),
        in_specs=[
            pl.BlockSpec((NC, _RB, D_IN), lambda i: (0, i, 0)),
            pl.BlockSpec((_RB, D_IN), lambda i: (i, 0)),
            pl.BlockSpec((NC, _RB, 16), lambda i: (0, i, 0)),
            pl.BlockSpec((D_IN, D_HID), lambda i: (0, 0)),
            pl.BlockSpec((D_HID,), lambda i: (0,)),
            pl.BlockSpec((D_HID, D_HID), lambda i: (0, 0)),
            pl.BlockSpec((D_HID,), lambda i: (0,)),
            pl.BlockSpec((D_HID, 1), lambda i: (0, 0)),
            pl.BlockSpec((1,), lambda i: (0,)),
        ],
        out_specs=pl.BlockSpec((_RB, 1), lambda i: (i, 0)),
        out_shape=jax.ShapeDtypeStruct((N, 1), jnp.float32),
    )(parts, y, deg16, W1, b1, W2, b2, W3, b3)


def kernel(input_embd, edge_index, W1, b1, W2, b2, W3, b3):
    edge_flat = edge_index.reshape(-1)
    zeros16 = jnp.zeros((N, 16), jnp.float32)
    zeros128 = jnp.zeros((N, D_IN), jnp.float32)
    deg16 = _degree_kernel(edge_flat, zeros16)
    y = _scale(deg16, input_embd)
    parts = _scatter_kernel(edge_flat, y, zeros128)
    return _mlp(parts, y, deg16, W1, b1, W2, b2, W3, b3)
